# baseline (speedup 1.0000x reference)
import functools

import jax
import jax.numpy as jnp
from jax import lax
from jax.experimental import pallas as pl
from jax.experimental.pallas import tpu as pltpu
from jax.experimental.pallas import tpu_sc as plsc

NC = 2
NS = 16
CHUNK = 64
NBUF = 4

CORE0_FRAC = 0.766


def _mm_body(x_ref, w_ref, o_ref):
    o_ref[...] = jnp.dot(x_ref[...], w_ref[...],
                         preferred_element_type=jnp.float32)


def _support_matmul(x, w):
    n, d_in = x.shape
    d_out = w.shape[1]
    blk = 1000
    return pl.pallas_call(
        _mm_body,
        grid=(n // blk,),
        in_specs=[
            pl.BlockSpec((blk, d_in), lambda i: (i, 0)),
            pl.BlockSpec((d_in, d_out), lambda i: (0, 0)),
        ],
        out_specs=pl.BlockSpec((blk, d_out), lambda i: (i, 0)),
        out_shape=jax.ShapeDtypeStruct((n, d_out), jnp.float32),
    )(x, w)


def _comb_body(p_ref, b_ref, o_ref):
    o_ref[...] = p_ref[0] + p_ref[1] + b_ref[...]


def _combine(partials, b, n):
    d = partials.shape[2]
    blk = 1000
    return pl.pallas_call(
        _comb_body,
        grid=(n // blk,),
        in_specs=[
            pl.BlockSpec((NC, blk, d), lambda i: (0, i, 0)),
            pl.BlockSpec((1, d), lambda i: (0, 0)),
        ],
        out_specs=pl.BlockSpec((blk, d), lambda i: (i, 0)),
        out_shape=jax.ShapeDtypeStruct((n, d), jnp.float32),
    )(partials, b.reshape(1, d))


def _split_chunks(total_chunks):
    per_s = -(-total_chunks // NS)
    nc0 = int(round(per_s * CORE0_FRAC / NBUF)) * NBUF
    nc1 = -(-(per_s - nc0) // NBUF) * NBUF
    while NS * (nc0 + nc1) < total_chunks:
        nc1 += NBUF
    return nc0, nc1


def _make_spmm(n_nodes, n_feat, nc0, nc1):
    n_pad = ((n_nodes + NS * 8 - 1) // (NS * 8)) * (NS * 8)
    rows_per_tile = n_pad // NS
    epw0 = nc0 * CHUNK
    epw1 = nc1 * CHUNK
    mesh = plsc.VectorSubcoreMesh(core_axis_name="c", subcore_axis_name="s",
                                  num_cores=NC, num_subcores=NS)

    @functools.partial(
        pl.kernel,
        out_type=jax.ShapeDtypeStruct((NC, n_pad, n_feat), jnp.float32),
        mesh=mesh,
        scratch_types=[
            pltpu.VMEM((epw0,), jnp.int32),
            pltpu.VMEM((NBUF, CHUNK), jnp.float32),
            pltpu.VMEM((NBUF, CHUNK), jnp.int32),
            pltpu.VMEM((NBUF, CHUNK, n_feat), jnp.float32),
            pltpu.VMEM_SHARED((n_pad, n_feat), jnp.float32),
        ] + [pltpu.SemaphoreType.DMA] * (2 * NBUF + 1),
    )
    def spmm(support_hbm, cols_hbm, rows_hbm, vals_hbm, out_hbm,
             cols_v, vals_b, rows_b, gath, acc, *sems):
        sem_g = sems[:NBUF]
        sem_s = sems[NBUF:2 * NBUF]
        sem_i = sems[2 * NBUF]
        cid = lax.axis_index("c")
        sid = lax.axis_index("s")
        row0 = sid * rows_per_tile
        base = jnp.where(cid == 0, sid * epw0, NS * epw0 + sid * epw1)
        my_quads = jnp.where(cid == 0, nc0 // NBUF, nc1 // NBUF)
        n_my_chunks = my_quads * NBUF

        @pl.when(cid == 0)
        def _():
            pltpu.async_copy(cols_hbm.at[pl.ds(base, epw0)],
                             cols_v.at[pl.ds(0, epw0)], sem_i)

        @pl.when(cid == 1)
        def _():
            pltpu.async_copy(cols_hbm.at[pl.ds(base, epw1)],
                             cols_v.at[pl.ds(0, epw1)], sem_i)

        zero16 = jnp.zeros((16,), jnp.float32)

        def _zbody(i, _):
            for j in range(n_feat // 16):
                gath[0, i, pl.ds(j * 16, 16)] = zero16
            return 0

        lax.fori_loop(0, CHUNK, _zbody, 0)
        full, rem = divmod(rows_per_tile, CHUNK)
        for i in range(full):
            pltpu.sync_copy(gath.at[0],
                            acc.at[pl.ds(row0 + i * CHUNK, CHUNK)])
        if rem:
            pltpu.sync_copy(gath.at[0, pl.ds(0, rem)],
                            acc.at[pl.ds(row0 + full * CHUNK, rem)])

        @pl.when(cid == 0)
        def _():
            pltpu.make_async_copy(cols_hbm.at[pl.ds(0, epw0)],
                                  cols_v.at[pl.ds(0, epw0)], sem_i).wait()

        @pl.when(cid == 1)
        def _():
            pltpu.make_async_copy(cols_hbm.at[pl.ds(0, epw1)],
                                  cols_v.at[pl.ds(0, epw1)], sem_i).wait()

        plsc.subcore_barrier()

        def _gather(k, b):
            off = base + k * CHUNK
            pltpu.async_copy(rows_hbm.at[pl.ds(off, CHUNK)], rows_b.at[b],
                             sem_g[b])
            pltpu.async_copy(vals_hbm.at[pl.ds(off, CHUNK)], vals_b.at[b],
                             sem_g[b])
            pltpu.async_copy(
                support_hbm.at[cols_v.at[pl.ds(k * CHUNK, CHUNK)]],
                gath.at[b], sem_g[b])

        def _wait_gather(b):
            pltpu.make_async_copy(rows_hbm.at[pl.ds(0, CHUNK)],
                                  rows_b.at[b], sem_g[b]).wait()
            pltpu.make_async_copy(vals_hbm.at[pl.ds(0, CHUNK)],
                                  vals_b.at[b], sem_g[b]).wait()
            pltpu.make_async_copy(support_hbm.at[pl.ds(0, CHUNK)],
                                  gath.at[b], sem_g[b]).wait()

        def _scatter(b):
            pltpu.async_copy(gath.at[b], acc.at[rows_b.at[b]], sem_s[b],
                             add=True)

        def _wait_scatter(b):
            pltpu.make_async_copy(gath.at[b], acc.at[pl.ds(0, CHUNK)],
                                  sem_s[b]).wait()

        _gather(0, 0)
        _gather(1, 1)

        def _pipe_body(kk, _):
            for b in range(NBUF):
                k = kk * NBUF + b
                b2 = (b + 1) % NBUF
                @pl.when(k < 0)
                def _():
                    _wait_scatter(b2)

                @pl.when(k + 2 < n_my_chunks)
                def _():
                    _gather(k + 2, (b + 2) % NBUF)

                _wait_gather(b)

                def _scale(g, _):
                    vv = vals_b[b, pl.ds(g * 16, 16)]
                    for lane in range(16):
                        v = vv[lane]
                        edge = g * 16 + lane
                        for j in range(n_feat // 16):
                            sl = pl.ds(j * 16, 16)
                            gath[b, edge, sl] = gath[b, edge, sl] * v
                    return 0

                @pl.when(k < 0)
                def _():
                    _scatter(b)
            return 0

        lax.fori_loop(0, my_quads, _pipe_body, 0)
        plsc.subcore_barrier()
        pltpu.sync_copy(acc.at[pl.ds(row0, rows_per_tile)],
                        out_hbm.at[cid, pl.ds(row0, rows_per_tile)])

    return spmm


def kernel(embeddings, edge_index, adj_values, W, b):
    n, d_in = embeddings.shape
    d_out = W.shape[1]
    e = edge_index.shape[1]

    support = _support_matmul(embeddings, W)

    nc0, nc1 = _split_chunks(-(-e // CHUNK))
    e_pad = NS * (nc0 + nc1) * CHUNK
    pad = e_pad - e
    rows = edge_index[0].astype(jnp.int32)
    cols = edge_index[1].astype(jnp.int32)
    vals = adj_values.astype(jnp.float32)
    if pad:
        rows = jnp.pad(rows, (0, pad))
        cols = jnp.pad(cols, (0, pad))
        vals = jnp.pad(vals, (0, pad))

    spmm = _make_spmm(n, d_out, nc0, nc1)
    partials = spmm(support, cols, rows, vals)
    return _combine(partials, b, n)


# speedup vs baseline: 1.0166x; 1.0166x over previous
import functools

import jax
import jax.numpy as jnp
from jax import lax
from jax.experimental import pallas as pl
from jax.experimental.pallas import tpu as pltpu
from jax.experimental.pallas import tpu_sc as plsc

NC = 2
NS = 16
CHUNK = 64
NBUF = 4

CORE0_FRAC = 0.766


def _mm_body(x_ref, w_ref, o_ref):
    o_ref[...] = jnp.dot(x_ref[...], w_ref[...],
                         preferred_element_type=jnp.float32)


def _support_matmul(x, w):
    n, d_in = x.shape
    d_out = w.shape[1]
    blk = 2000
    return pl.pallas_call(
        _mm_body,
        grid=(n // blk,),
        in_specs=[
            pl.BlockSpec((blk, d_in), lambda i: (i, 0)),
            pl.BlockSpec((d_in, d_out), lambda i: (0, 0)),
        ],
        out_specs=pl.BlockSpec((blk, d_out), lambda i: (i, 0)),
        out_shape=jax.ShapeDtypeStruct((n, d_out), jnp.float32),
    )(x, w)


def _comb_body(p_ref, b_ref, o_ref):
    o_ref[...] = p_ref[0] + p_ref[1] + b_ref[...]


def _combine(partials, b, n):
    d = partials.shape[2]
    blk = 2000
    return pl.pallas_call(
        _comb_body,
        grid=(n // blk,),
        in_specs=[
            pl.BlockSpec((NC, blk, d), lambda i: (0, i, 0)),
            pl.BlockSpec((1, d), lambda i: (0, 0)),
        ],
        out_specs=pl.BlockSpec((blk, d), lambda i: (i, 0)),
        out_shape=jax.ShapeDtypeStruct((n, d), jnp.float32),
    )(partials, b.reshape(1, d))


def _split_chunks(total_chunks):
    per_s = -(-total_chunks // NS)
    nc0 = int(round(per_s * CORE0_FRAC / NBUF)) * NBUF
    nc1 = -(-(per_s - nc0) // NBUF) * NBUF
    while NS * (nc0 + nc1) < total_chunks:
        nc1 += NBUF
    return nc0, nc1


def _make_spmm(n_nodes, n_feat, nc0, nc1):
    n_pad = ((n_nodes + NS * 8 - 1) // (NS * 8)) * (NS * 8)
    rows_per_tile = n_pad // NS
    epw0 = nc0 * CHUNK
    epw1 = nc1 * CHUNK
    mesh = plsc.VectorSubcoreMesh(core_axis_name="c", subcore_axis_name="s",
                                  num_cores=NC, num_subcores=NS)

    @functools.partial(
        pl.kernel,
        out_type=jax.ShapeDtypeStruct((NC, n_pad, n_feat), jnp.float32),
        mesh=mesh,
        scratch_types=[
            pltpu.VMEM((epw0,), jnp.int32),
            pltpu.VMEM((NBUF, CHUNK), jnp.float32),
            pltpu.VMEM((NBUF, CHUNK), jnp.int32),
            pltpu.VMEM((NBUF, CHUNK, n_feat), jnp.float32),
            pltpu.VMEM_SHARED((n_pad, n_feat), jnp.float32),
        ] + [pltpu.SemaphoreType.DMA] * (2 * NBUF + 1),
    )
    def spmm(support_hbm, cols_hbm, rows_hbm, vals_hbm, out_hbm,
             cols_v, vals_b, rows_b, gath, acc, *sems):
        sem_g = sems[:NBUF]
        sem_s = sems[NBUF:2 * NBUF]
        sem_i = sems[2 * NBUF]
        cid = lax.axis_index("c")
        sid = lax.axis_index("s")
        row0 = sid * rows_per_tile
        base = jnp.where(cid == 0, sid * epw0, NS * epw0 + sid * epw1)
        my_quads = jnp.where(cid == 0, nc0 // NBUF, nc1 // NBUF)
        n_my_chunks = my_quads * NBUF

        @pl.when(cid == 0)
        def _():
            pltpu.async_copy(cols_hbm.at[pl.ds(base, epw0)],
                             cols_v.at[pl.ds(0, epw0)], sem_i)

        @pl.when(cid == 1)
        def _():
            pltpu.async_copy(cols_hbm.at[pl.ds(base, epw1)],
                             cols_v.at[pl.ds(0, epw1)], sem_i)

        zero16 = jnp.zeros((16,), jnp.float32)

        def _zbody(i, _):
            for j in range(n_feat // 16):
                gath[0, i, pl.ds(j * 16, 16)] = zero16
            return 0

        lax.fori_loop(0, CHUNK, _zbody, 0)
        full, rem = divmod(rows_per_tile, CHUNK)
        for i in range(full):
            pltpu.sync_copy(gath.at[0],
                            acc.at[pl.ds(row0 + i * CHUNK, CHUNK)])
        if rem:
            pltpu.sync_copy(gath.at[0, pl.ds(0, rem)],
                            acc.at[pl.ds(row0 + full * CHUNK, rem)])

        @pl.when(cid == 0)
        def _():
            pltpu.make_async_copy(cols_hbm.at[pl.ds(0, epw0)],
                                  cols_v.at[pl.ds(0, epw0)], sem_i).wait()

        @pl.when(cid == 1)
        def _():
            pltpu.make_async_copy(cols_hbm.at[pl.ds(0, epw1)],
                                  cols_v.at[pl.ds(0, epw1)], sem_i).wait()

        plsc.subcore_barrier()

        def _gather(k, b):
            off = base + k * CHUNK
            pltpu.async_copy(rows_hbm.at[pl.ds(off, CHUNK)], rows_b.at[b],
                             sem_g[b])
            pltpu.async_copy(vals_hbm.at[pl.ds(off, CHUNK)], vals_b.at[b],
                             sem_g[b])
            pltpu.async_copy(
                support_hbm.at[cols_v.at[pl.ds(k * CHUNK, CHUNK)]],
                gath.at[b], sem_g[b])

        def _wait_gather(b):
            pltpu.make_async_copy(rows_hbm.at[pl.ds(0, CHUNK)],
                                  rows_b.at[b], sem_g[b]).wait()
            pltpu.make_async_copy(vals_hbm.at[pl.ds(0, CHUNK)],
                                  vals_b.at[b], sem_g[b]).wait()
            pltpu.make_async_copy(support_hbm.at[pl.ds(0, CHUNK)],
                                  gath.at[b], sem_g[b]).wait()

        def _scatter(b):
            pltpu.async_copy(gath.at[b], acc.at[rows_b.at[b]], sem_s[b],
                             add=True)

        def _wait_scatter(b):
            pltpu.make_async_copy(gath.at[b], acc.at[pl.ds(0, CHUNK)],
                                  sem_s[b]).wait()

        _gather(0, 0)

        def _pipe_body(kk, _):
            for b in range(NBUF):
                k = kk * NBUF + b
                b2 = (b + 1) % NBUF
                @pl.when(k >= NBUF - 1)
                def _():
                    _wait_scatter(b2)

                @pl.when(k + 1 < n_my_chunks)
                def _():
                    _gather(k + 1, b2)

                _wait_gather(b)

                def _scale(g, _):
                    vv = vals_b[b, pl.ds(g * 16, 16)]
                    for lane in range(16):
                        v = vv[lane]
                        edge = g * 16 + lane
                        for j in range(n_feat // 16):
                            sl = pl.ds(j * 16, 16)
                            gath[b, edge, sl] = gath[b, edge, sl] * v
                    return 0

                lax.fori_loop(0, CHUNK // 16, _scale, 0)
                _scatter(b)
            return 0

        lax.fori_loop(0, my_quads, _pipe_body, 0)
        _wait_scatter(1)
        _wait_scatter(2)
        _wait_scatter(3)
        plsc.subcore_barrier()
        pltpu.sync_copy(acc.at[pl.ds(row0, rows_per_tile)],
                        out_hbm.at[cid, pl.ds(row0, rows_per_tile)])

    return spmm


def kernel(embeddings, edge_index, adj_values, W, b):
    n, d_in = embeddings.shape
    d_out = W.shape[1]
    e = edge_index.shape[1]

    support = _support_matmul(embeddings, W)

    nc0, nc1 = _split_chunks(-(-e // CHUNK))
    e_pad = NS * (nc0 + nc1) * CHUNK
    pad = e_pad - e
    rows = edge_index[0].astype(jnp.int32)
    cols = edge_index[1].astype(jnp.int32)
    vals = adj_values.astype(jnp.float32)
    if pad:
        rows = jnp.pad(rows, (0, pad))
        cols = jnp.pad(cols, (0, pad))
        vals = jnp.pad(vals, (0, pad))

    spmm = _make_spmm(n, d_out, nc0, nc1)
    partials = spmm(support, cols, rows, vals)
    return _combine(partials, b, n)
